# baseline (speedup 1.0000x reference)
import functools

import jax
import jax.numpy as jnp
from jax import lax
from jax.experimental import pallas as pl
from jax.experimental.pallas import tpu as pltpu
from jax.experimental.pallas import tpu_sc as plsc

_N = 10000
_D = 128
_K = 32
_BR = 128
_NBLK = 79
_NPAD = _NBLK * _BR
_DIN = 136

_F32 = jnp.float32
_I32 = jnp.int32

_SC_NC = 2
_SC_NS = 16
_SC_NW = _SC_NC * _SC_NS
_EDGES = _N * _K
_E_PER_W = _EDGES // _SC_NW
_GCHUNK = 80
_GITERS = _E_PER_W // _GCHUNK


def _stageA_body(xp_ref, wc_ref, bvec_ref, g_ref, cq_ref, xmax_ref):
    i = pl.program_id(0)
    xb = xp_ref[...]
    y = lax.dot_general(xb, wc_ref[...], (((1,), (0,)), ((), ())),
                        precision=lax.Precision.HIGHEST)
    y = y + bvec_ref[...]
    g_ref[...] = y[:, :256]
    cq_ref[...] = y[:, 256:512]
    row = i * _BR + lax.broadcasted_iota(_I32, (_BR, 1), 0)
    xm = jnp.where(row < _N, xb[:, :_D], -jnp.inf)
    bmax = jnp.max(xm, axis=0, keepdims=True)

    @pl.when(i == 0)
    def _():
        xmax_ref[...] = jnp.full((1, _D), -jnp.inf, _F32)

    xmax_ref[...] = jnp.maximum(xmax_ref[...], bmax)


def _stageA(xp_pad, wc, bvec):
    return pl.pallas_call(
        _stageA_body,
        grid=(_NBLK,),
        in_specs=[
            pl.BlockSpec((_BR, _DIN), lambda i: (i, 0)),
            pl.BlockSpec((_DIN, 512), lambda i: (0, 0)),
            pl.BlockSpec((1, 512), lambda i: (0, 0)),
        ],
        out_specs=[
            pl.BlockSpec((_BR, 256), lambda i: (i, 0)),
            pl.BlockSpec((_BR, 256), lambda i: (i, 0)),
            pl.BlockSpec((1, _D), lambda i: (0, 0)),
        ],
        out_shape=[
            jax.ShapeDtypeStruct((_N, 256), _F32),
            jax.ShapeDtypeStruct((_N, 256), _F32),
            jax.ShapeDtypeStruct((1, _D), _F32),
        ],
    )(xp_pad, wc, bvec)


_TPG = 16
_NG = 5
_NPC = _NG * _TPG * 128
_NCLS = _NG * 128
_BIGI = 2**30


def _stageB_body(xr_ref, xt_ref, idx_ref, dist_ref,
                 m1, m2, m3, c1, c2, c3, lv, li):
    i = pl.program_id(0)
    xr = xr_ref[...]
    xt = xt_ref[...]
    sqi = jnp.sum(xr * xr, axis=1, keepdims=True)
    sqj = jnp.sum(xt * xt, axis=0, keepdims=True)
    mm = lax.dot_general(xr, xt, (((1,), (0,)), ((), ())))
    col = lax.broadcasted_iota(_I32, (_BR, _NPC), 1)
    row = i * _BR + lax.broadcasted_iota(_I32, (_BR, _NPC), 0)
    dist = (sqi + sqj) - 2.0 * mm
    dist = dist + jnp.where(col == row, _F32(1e10), _F32(0.0))
    dist = jnp.where(col >= _N, jnp.inf, dist)
    dist_ref[...] = dist

    lv[...] = jnp.full((_BR, _NCLS), -jnp.inf, _F32)
    li[...] = jnp.full((_BR, _NCLS), -1, _I32)

    tl = lax.broadcasted_iota(_I32, (_BR, 128), 1)
    inf_t = jnp.full((_BR, 128), jnp.inf, _F32)
    big_t = jnp.full((_BR, 128), _BIGI, _I32)

    def build():
        lvv = lv[...]
        liv = li[...]
        for g in range(_NG):
            sl = pl.ds(g * 128, 128)
            tlv = lvv[:, g * 128:(g + 1) * 128]
            tli = liv[:, g * 128:(g + 1) * 128]
            outs = []
            for _rnd in range(3):
                m = inf_t
                for t in range(_TPG):
                    ti = g * _TPG + t
                    tile = dist_ref[:, ti * 128:(ti + 1) * 128]
                    valid = (tile > tlv) | ((tile == tlv) & (tl + ti * 128 > tli))
                    m = jnp.minimum(m, jnp.where(valid, tile, jnp.inf))
                c = big_t
                for t in range(_TPG):
                    ti = g * _TPG + t
                    tile = dist_ref[:, ti * 128:(ti + 1) * 128]
                    colt = tl + ti * 128
                    valid = (tile > tlv) | ((tile == tlv) & (colt > tli))
                    c = jnp.minimum(c, jnp.where(valid & (tile == m), colt, _BIGI))
                outs.append((m, c))
                tlv, tli = m, c
            m1[:, sl] = outs[0][0]
            c1[:, sl] = outs[0][1]
            m2[:, sl] = outs[1][0]
            c2[:, sl] = outs[1][1]
            m3[:, sl] = outs[2][0]
            c3[:, sl] = outs[2][1]

    build()

    clsio = lax.broadcasted_iota(_I32, (_BR, _NCLS), 1)
    flag = jnp.bool_(False)
    for k in range(_K):
        @pl.when(flag)
        def _():
            build()

        a1 = m1[...]
        b1 = c1[...]
        mrow = jnp.min(a1, axis=1, keepdims=True)
        j = jnp.min(jnp.where(a1 == mrow, b1, _BIGI), axis=1, keepdims=True)
        idx_ref[:, pl.ds(k, 1)] = j
        cls = lax.shift_right_logical(j, 11) * 128 + (j & 127)
        iscls = clsio == cls
        a2 = m2[...]
        a3 = m3[...]
        b2 = c2[...]
        b3 = c3[...]
        flag = jnp.max(jnp.where(iscls & (a2 == jnp.inf), 1, 0)) > 0
        lv[...] = jnp.where(iscls, mrow, lv[...])
        li[...] = jnp.where(iscls, j, li[...])
        m1[...] = jnp.where(iscls, a2, a1)
        m2[...] = jnp.where(iscls, a3, a2)
        m3[...] = jnp.where(iscls, jnp.inf, a3)
        c1[...] = jnp.where(iscls, b2, b1)
        c2[...] = jnp.where(iscls, b3, b2)
        c3[...] = jnp.where(iscls, _BIGI, b3)


def _stageB(xpad, xt):
    return pl.pallas_call(
        _stageB_body,
        grid=(_NBLK,),
        in_specs=[
            pl.BlockSpec((_BR, _D), lambda i: (i, 0)),
            pl.BlockSpec((_D, _NPC), lambda i: (0, 0)),
        ],
        out_specs=pl.BlockSpec((_BR, _K), lambda i: (i, 0)),
        out_shape=jax.ShapeDtypeStruct((_NPAD, _K), _I32),
        scratch_shapes=[
            pltpu.VMEM((_BR, _NPC), _F32),
            pltpu.VMEM((_BR, _NCLS), _F32),
            pltpu.VMEM((_BR, _NCLS), _F32),
            pltpu.VMEM((_BR, _NCLS), _F32),
            pltpu.VMEM((_BR, _NCLS), _I32),
            pltpu.VMEM((_BR, _NCLS), _I32),
            pltpu.VMEM((_BR, _NCLS), _I32),
            pltpu.VMEM((_BR, _NCLS), _F32),
            pltpu.VMEM((_BR, _NCLS), _I32),
        ],
    )(xpad, xt)


def _stageC(table, idx):
    mesh = plsc.VectorSubcoreMesh(core_axis_name="c", subcore_axis_name="s")

    @functools.partial(
        pl.kernel,
        mesh=mesh,
        out_type=jax.ShapeDtypeStruct((_EDGES, 256), _F32),
        scratch_types=[
            pltpu.VMEM((_E_PER_W,), _I32),
            pltpu.VMEM((_GCHUNK, 256), _F32),
            pltpu.SemaphoreType.DMA,
        ],
    )
    def k(table_hbm, idx_hbm, out_hbm, idx_v, rows_v, sem):
        wid = lax.axis_index("s") * _SC_NC + lax.axis_index("c")
        base = wid * _E_PER_W
        pltpu.sync_copy(idx_hbm.at[pl.ds(base, _E_PER_W)], idx_v)

        def body(t, _):
            off = pl.multiple_of(t * _GCHUNK, 8)
            pltpu.async_copy(
                table_hbm.at[idx_v.at[pl.ds(off, _GCHUNK)]], rows_v, sem
            ).wait()
            pltpu.sync_copy(rows_v, out_hbm.at[pl.ds(base + off, _GCHUNK)])
            return ()

        lax.fori_loop(0, _GITERS, body, (), unroll=False)

    return k(table, idx)


def _stageD_body(gath_ref, cq_ref, g_ref, xmax_ref, w1_ref, w2_ref, bm_ref,
                 out_ref):
    cq = cq_ref[...]
    g = g_ref[...]
    c = cq[:, :_D]
    q = cq[:, _D:]
    s_own = g[:, :_D]
    u_own = g[:, _D:]

    a_self = c - s_own
    m = a_self
    for k in range(_K):
        m = jnp.maximum(m, c - gath_ref[:, k * 256:k * 256 + _D])
    es = jnp.exp(a_self - m)
    den = es
    num = es * u_own
    for k in range(_K):
        blk = gath_ref[:, pl.ds(k * 256, 256)]
        e = jnp.exp((c - blk[:, :_D]) - m)
        den = den + e
        num = num + e * blk[:, _D:]
    h1 = (num + den * q) / (den + _F32(1e-16))

    r = lax.dot_general(xmax_ref[...], w2_ref[...], (((1,), (0,)), ((), ())),
                        precision=lax.Precision.HIGHEST) + bm_ref[...]
    o = lax.dot_general(h1, w1_ref[...], (((1,), (0,)), ((), ())),
                        precision=lax.Precision.HIGHEST) + r
    out_ref[...] = jnp.maximum(o, 0.0)


def _stageD(gath2, cq, g, xmax, w1, w2, bm):
    return pl.pallas_call(
        _stageD_body,
        grid=(_NBLK,),
        in_specs=[
            pl.BlockSpec((_BR, _K * 256), lambda i: (i, 0)),
            pl.BlockSpec((_BR, 256), lambda i: (i, 0)),
            pl.BlockSpec((_BR, 256), lambda i: (i, 0)),
            pl.BlockSpec((1, _D), lambda i: (0, 0)),
            pl.BlockSpec((_D, _D), lambda i: (0, 0)),
            pl.BlockSpec((_D, _D), lambda i: (0, 0)),
            pl.BlockSpec((1, _D), lambda i: (0, 0)),
        ],
        out_specs=pl.BlockSpec((_BR, _D), lambda i: (i, 0)),
        out_shape=jax.ShapeDtypeStruct((_N, _D), _F32),
    )(gath2, cq, g, xmax, w1, w2, bm)


def kernel(x, pos, W_lin, W_src, W_dst, W_pos, b_pos, W_mlp, b_mlp):
    xpad = jnp.zeros((_NPAD, _D), _F32).at[:_N].set(x)
    xt = jnp.zeros((_D, _NPC), _F32).at[:, :_N].set(x.T)

    pz = jnp.zeros((_DIN - _D - 3, 512), _F32)
    wx = jnp.concatenate([W_src, W_lin, W_dst, jnp.zeros((_D, _D), _F32)],
                         axis=1)
    wp = jnp.concatenate([W_pos, -W_pos, W_pos, W_pos], axis=1)
    wc = jnp.concatenate([wx, wp, pz], axis=0)
    zb = jnp.zeros((256,), _F32)
    bvec = jnp.concatenate([zb, b_pos, b_pos]).reshape(1, 512)

    xp_pad = jnp.zeros((_NPAD, _DIN), _F32)
    xp_pad = xp_pad.at[:_N, :_D].set(x).at[:_N, _D:_D + 3].set(pos)

    g, cq, xmax = _stageA(xp_pad, wc, bvec)
    idx = _stageB(xpad, xt)[:_N, :_K].reshape(-1)
    gath = _stageC(g, idx)
    gath2 = gath.reshape(_N, _K * 256)

    w1 = W_mlp[:_D, :]
    w2 = W_mlp[_D:, :]
    bm = b_mlp.reshape(1, _D)
    return _stageD(gath2, cq, g, xmax, w1, w2, bm)


# speedup vs baseline: 5.1192x; 5.1192x over previous
import functools

import jax
import jax.numpy as jnp
from jax import lax
from jax.experimental import pallas as pl
from jax.experimental.pallas import tpu as pltpu
from jax.experimental.pallas import tpu_sc as plsc

_N = 10000
_D = 128
_K = 32
_BR = 128
_NBLK = 79
_NPAD = _NBLK * _BR
_DIN = 136

_F32 = jnp.float32
_I32 = jnp.int32

_SC_NC = 2
_SC_NS = 16
_SC_NW = _SC_NC * _SC_NS
_EDGES = _N * _K
_E_PER_W = _EDGES // _SC_NW
_GCHUNK = 80
_GITERS = _E_PER_W // _GCHUNK


def _stageA_body(xp_ref, wc_ref, bvec_ref, g_ref, cq_ref, xmax_ref):
    i = pl.program_id(0)
    xb = xp_ref[...]
    y = lax.dot_general(xb, wc_ref[...], (((1,), (0,)), ((), ())),
                        precision=lax.Precision.HIGHEST)
    y = y + bvec_ref[...]
    g_ref[...] = y[:, :256]
    cq_ref[...] = y[:, 256:512]
    row = i * _BR + lax.broadcasted_iota(_I32, (_BR, 1), 0)
    xm = jnp.where(row < _N, xb[:, :_D], -jnp.inf)
    bmax = jnp.max(xm, axis=0, keepdims=True)

    @pl.when(i == 0)
    def _():
        xmax_ref[...] = jnp.full((1, _D), -jnp.inf, _F32)

    xmax_ref[...] = jnp.maximum(xmax_ref[...], bmax)


def _stageA(xp_pad, wc, bvec):
    return pl.pallas_call(
        _stageA_body,
        grid=(_NBLK,),
        in_specs=[
            pl.BlockSpec((_BR, _DIN), lambda i: (i, 0)),
            pl.BlockSpec((_DIN, 512), lambda i: (0, 0)),
            pl.BlockSpec((1, 512), lambda i: (0, 0)),
        ],
        out_specs=[
            pl.BlockSpec((_BR, 256), lambda i: (i, 0)),
            pl.BlockSpec((_BR, 256), lambda i: (i, 0)),
            pl.BlockSpec((1, _D), lambda i: (0, 0)),
        ],
        out_shape=[
            jax.ShapeDtypeStruct((_N, 256), _F32),
            jax.ShapeDtypeStruct((_N, 256), _F32),
            jax.ShapeDtypeStruct((1, _D), _F32),
        ],
    )(xp_pad, wc, bvec)


_TPG = 16
_NG = 5
_NPC = _NG * _TPG * 128
_NCLS = _NG * 128
_BIGI = 2**30


_NSLOT = 4


def _stageB_body(xr_ref, xt_ref, idx_ref, dist_ref, ms, cs):
    i = pl.program_id(0)
    xr = xr_ref[...]
    xt = xt_ref[...]
    sqi = jnp.sum(xr * xr, axis=1, keepdims=True)
    sqj = jnp.sum(xt * xt, axis=0, keepdims=True)
    mm = lax.dot_general(xr, xt, (((1,), (0,)), ((), ())))
    col = lax.broadcasted_iota(_I32, (_BR, _NPC), 1)
    row = i * _BR + lax.broadcasted_iota(_I32, (_BR, _NPC), 0)
    dist = (sqi + sqj) - 2.0 * mm
    dist = dist + jnp.where(col == row, _F32(1e10), _F32(0.0))
    dist = jnp.where(col >= _N, jnp.inf, dist)
    dist_ref[...] = dist

    tl = lax.broadcasted_iota(_I32, (_BR, 128), 1)
    inf_t = jnp.full((_BR, 128), jnp.inf, _F32)
    big_t = jnp.full((_BR, 128), _BIGI, _I32)

    for g in range(_NG):
        sl = pl.ds(g * 128, 128)
        tlv = jnp.full((_BR, 128), -jnp.inf, _F32)
        tli = jnp.full((_BR, 128), -1, _I32)
        for rnd in range(_NSLOT):
            m = inf_t
            for t in range(_TPG):
                ti = g * _TPG + t
                tile = dist_ref[:, ti * 128:(ti + 1) * 128]
                valid = (tile > tlv) | ((tile == tlv) & (tl + ti * 128 > tli))
                m = jnp.minimum(m, jnp.where(valid, tile, jnp.inf))
            c = big_t
            for t in range(_TPG):
                ti = g * _TPG + t
                tile = dist_ref[:, ti * 128:(ti + 1) * 128]
                colt = tl + ti * 128
                valid = (tile > tlv) | ((tile == tlv) & (colt > tli))
                c = jnp.minimum(c, jnp.where(valid & (tile == m), colt, _BIGI))
            ms[rnd][:, sl] = m
            cs[rnd][:, sl] = c
            tlv, tli = m, c

    clsio = lax.broadcasted_iota(_I32, (_BR, _NCLS), 1)
    overflow = jnp.zeros((_BR, _NCLS), jnp.bool_)
    for k in range(_K):
        a = [ms[r][...] for r in range(_NSLOT)]
        b = [cs[r][...] for r in range(_NSLOT)]
        mrow = jnp.min(a[0], axis=1, keepdims=True)
        j = jnp.min(jnp.where(a[0] == mrow, b[0], _BIGI), axis=1,
                    keepdims=True)
        idx_ref[:, pl.ds(k, 1)] = j
        cls = lax.shift_right_logical(j, 11) * 128 + (j & 127)
        iscls = clsio == cls
        overflow = overflow | (iscls & (a[1] == jnp.inf))
        for r in range(_NSLOT - 1):
            ms[r][...] = jnp.where(iscls, a[r + 1], a[r])
            cs[r][...] = jnp.where(iscls, b[r + 1], b[r])
        ms[_NSLOT - 1][...] = jnp.where(iscls, jnp.inf, a[_NSLOT - 1])
        cs[_NSLOT - 1][...] = jnp.where(iscls, _BIGI, b[_NSLOT - 1])

    bad = jnp.max(overflow.astype(_I32)) > 0

    @pl.when(bad)
    def _():
        lvf = jnp.full((_BR, 1), -jnp.inf, _F32)
        lif = jnp.full((_BR, 1), -1, _I32)
        for k in range(_K):
            d = dist_ref[...]
            validf = (d > lvf) | ((d == lvf) & (col > lif))
            candf = jnp.where(validf, d, jnp.inf)
            mf = jnp.min(candf, axis=1, keepdims=True)
            jf = jnp.min(jnp.where(candf == mf, col, _BIGI), axis=1,
                         keepdims=True)
            idx_ref[:, pl.ds(k, 1)] = jf
            lvf, lif = mf, jf


def _stageB(xpad, xt):
    def body(xr_ref, xt_ref, idx_ref, *scratch):
        ms = list(scratch[1:1 + _NSLOT])
        cs = list(scratch[1 + _NSLOT:])
        _stageB_body(xr_ref, xt_ref, idx_ref, scratch[0], ms, cs)

    return pl.pallas_call(
        body,
        grid=(_NBLK,),
        in_specs=[
            pl.BlockSpec((_BR, _D), lambda i: (i, 0)),
            pl.BlockSpec((_D, _NPC), lambda i: (0, 0)),
        ],
        out_specs=pl.BlockSpec((_BR, _K), lambda i: (i, 0)),
        out_shape=jax.ShapeDtypeStruct((_NPAD, _K), _I32),
        scratch_shapes=(
            [pltpu.VMEM((_BR, _NPC), _F32)]
            + [pltpu.VMEM((_BR, _NCLS), _F32) for _ in range(_NSLOT)]
            + [pltpu.VMEM((_BR, _NCLS), _I32) for _ in range(_NSLOT)]
        ),
    )(xpad, xt)


def _stageC(table, idx):
    mesh = plsc.VectorSubcoreMesh(core_axis_name="c", subcore_axis_name="s")

    @functools.partial(
        pl.kernel,
        mesh=mesh,
        out_type=jax.ShapeDtypeStruct((_EDGES, 256), _F32),
        scratch_types=[
            pltpu.VMEM((_E_PER_W,), _I32),
            pltpu.VMEM((_GCHUNK, 256), _F32),
            pltpu.SemaphoreType.DMA,
        ],
    )
    def k(table_hbm, idx_hbm, out_hbm, idx_v, rows_v, sem):
        wid = lax.axis_index("s") * _SC_NC + lax.axis_index("c")
        base = wid * _E_PER_W
        pltpu.sync_copy(idx_hbm.at[pl.ds(base, _E_PER_W)], idx_v)

        def body(t, _):
            off = pl.multiple_of(t * _GCHUNK, 8)
            pltpu.async_copy(
                table_hbm.at[idx_v.at[pl.ds(off, _GCHUNK)]], rows_v, sem
            ).wait()
            pltpu.sync_copy(rows_v, out_hbm.at[pl.ds(base + off, _GCHUNK)])
            return ()

        lax.fori_loop(0, _GITERS, body, (), unroll=False)

    return k(table, idx)


def _stageD_body(gath_ref, cq_ref, g_ref, xmax_ref, w1_ref, w2_ref, bm_ref,
                 out_ref):
    cq = cq_ref[...]
    g = g_ref[...]
    c = cq[:, :_D]
    q = cq[:, _D:]
    s_own = g[:, :_D]
    u_own = g[:, _D:]

    a_self = c - s_own
    m = a_self
    for k in range(_K):
        m = jnp.maximum(m, c - gath_ref[:, k * 256:k * 256 + _D])
    es = jnp.exp(a_self - m)
    den = es
    num = es * u_own
    for k in range(_K):
        blk = gath_ref[:, pl.ds(k * 256, 256)]
        e = jnp.exp((c - blk[:, :_D]) - m)
        den = den + e
        num = num + e * blk[:, _D:]
    h1 = (num + den * q) / (den + _F32(1e-16))

    r = lax.dot_general(xmax_ref[...], w2_ref[...], (((1,), (0,)), ((), ())),
                        precision=lax.Precision.HIGHEST) + bm_ref[...]
    o = lax.dot_general(h1, w1_ref[...], (((1,), (0,)), ((), ())),
                        precision=lax.Precision.HIGHEST) + r
    out_ref[...] = jnp.maximum(o, 0.0)


def _stageD(gath2, cq, g, xmax, w1, w2, bm):
    return pl.pallas_call(
        _stageD_body,
        grid=(_NBLK,),
        in_specs=[
            pl.BlockSpec((_BR, _K * 256), lambda i: (i, 0)),
            pl.BlockSpec((_BR, 256), lambda i: (i, 0)),
            pl.BlockSpec((_BR, 256), lambda i: (i, 0)),
            pl.BlockSpec((1, _D), lambda i: (0, 0)),
            pl.BlockSpec((_D, _D), lambda i: (0, 0)),
            pl.BlockSpec((_D, _D), lambda i: (0, 0)),
            pl.BlockSpec((1, _D), lambda i: (0, 0)),
        ],
        out_specs=pl.BlockSpec((_BR, _D), lambda i: (i, 0)),
        out_shape=jax.ShapeDtypeStruct((_N, _D), _F32),
    )(gath2, cq, g, xmax, w1, w2, bm)


def kernel(x, pos, W_lin, W_src, W_dst, W_pos, b_pos, W_mlp, b_mlp):
    xpad = jnp.zeros((_NPAD, _D), _F32).at[:_N].set(x)
    xt = jnp.zeros((_D, _NPC), _F32).at[:, :_N].set(x.T)

    pz = jnp.zeros((_DIN - _D - 3, 512), _F32)
    wx = jnp.concatenate([W_src, W_lin, W_dst, jnp.zeros((_D, _D), _F32)],
                         axis=1)
    wp = jnp.concatenate([W_pos, -W_pos, W_pos, W_pos], axis=1)
    wc = jnp.concatenate([wx, wp, pz], axis=0)
    zb = jnp.zeros((256,), _F32)
    bvec = jnp.concatenate([zb, b_pos, b_pos]).reshape(1, 512)

    xp_pad = jnp.zeros((_NPAD, _DIN), _F32)
    xp_pad = xp_pad.at[:_N, :_D].set(x).at[:_N, _D:_D + 3].set(pos)

    g, cq, xmax = _stageA(xp_pad, wc, bvec)
    idx = _stageB(xpad, xt)[:_N, :_K].reshape(-1)
    gath = _stageC(g, idx)
    gath2 = gath.reshape(_N, _K * 256)

    w1 = W_mlp[:_D, :]
    w2 = W_mlp[_D:, :]
    bm = b_mlp.reshape(1, _D)
    return _stageD(gath2, cq, g, xmax, w1, w2, bm)


# speedup vs baseline: 5.7581x; 1.1248x over previous
import functools

import jax
import jax.numpy as jnp
from jax import lax
from jax.experimental import pallas as pl
from jax.experimental.pallas import tpu as pltpu
from jax.experimental.pallas import tpu_sc as plsc

_N = 10000
_D = 128
_K = 32
_BR = 128
_NBLK = 79
_NPAD = _NBLK * _BR
_DIN = 136

_F32 = jnp.float32
_I32 = jnp.int32

_SC_NC = 2
_SC_NS = 16
_SC_NW = _SC_NC * _SC_NS
_EDGES = _N * _K
_E_PER_W = _EDGES // _SC_NW
_GCHUNK = 80
_GITERS = _E_PER_W // _GCHUNK


def _stageA_body(xp_ref, wc_ref, bvec_ref, g_ref, cq_ref, xmax_ref):
    i = pl.program_id(0)
    xb = xp_ref[...]
    y = lax.dot_general(xb, wc_ref[...], (((1,), (0,)), ((), ())),
                        precision=lax.Precision.HIGHEST)
    y = y + bvec_ref[...]
    g_ref[...] = y[:, :256]
    cq_ref[...] = y[:, 256:512]
    row = i * _BR + lax.broadcasted_iota(_I32, (_BR, 1), 0)
    xm = jnp.where(row < _N, xb[:, :_D], -jnp.inf)
    bmax = jnp.max(xm, axis=0, keepdims=True)

    @pl.when(i == 0)
    def _():
        xmax_ref[...] = jnp.full((1, _D), -jnp.inf, _F32)

    xmax_ref[...] = jnp.maximum(xmax_ref[...], bmax)


def _stageA(xp_pad, wc, bvec):
    return pl.pallas_call(
        _stageA_body,
        grid=(_NBLK,),
        in_specs=[
            pl.BlockSpec((_BR, _DIN), lambda i: (i, 0)),
            pl.BlockSpec((_DIN, 512), lambda i: (0, 0)),
            pl.BlockSpec((1, 512), lambda i: (0, 0)),
        ],
        out_specs=[
            pl.BlockSpec((_BR, 256), lambda i: (i, 0)),
            pl.BlockSpec((_BR, 256), lambda i: (i, 0)),
            pl.BlockSpec((1, _D), lambda i: (0, 0)),
        ],
        out_shape=[
            jax.ShapeDtypeStruct((_N, 256), _F32),
            jax.ShapeDtypeStruct((_N, 256), _F32),
            jax.ShapeDtypeStruct((1, _D), _F32),
        ],
    )(xp_pad, wc, bvec)


_TPG = 40
_NG = 2
_NPC = _NG * _TPG * 128
_NCLS = _NG * 128
_BIGI = 2**30


_NSLOT = 5


def _stageB_body(xr_ref, xt_ref, idx_ref, dist_ref, ms, cs):
    i = pl.program_id(0)
    xr = xr_ref[...]
    xt = xt_ref[...]
    sqi = jnp.sum(xr * xr, axis=1, keepdims=True)
    sqj = jnp.sum(xt * xt, axis=0, keepdims=True)
    mm = lax.dot_general(xr, xt, (((1,), (0,)), ((), ())))
    col = lax.broadcasted_iota(_I32, (_BR, _NPC), 1)
    row = i * _BR + lax.broadcasted_iota(_I32, (_BR, _NPC), 0)
    dist = (sqi + sqj) - 2.0 * mm
    dist = dist + jnp.where(col == row, _F32(1e10), _F32(0.0))
    dist = jnp.where(col >= _N, jnp.inf, dist)
    dist_ref[...] = dist

    tl = lax.broadcasted_iota(_I32, (_BR, 128), 1)
    inf_t = jnp.full((_BR, 128), jnp.inf, _F32)
    big_t = jnp.full((_BR, 128), _BIGI, _I32)

    for g in range(_NG):
        sl = pl.ds(g * 128, 128)
        tlv = jnp.full((_BR, 128), -jnp.inf, _F32)
        tli = jnp.full((_BR, 128), -1, _I32)
        for rnd in range(_NSLOT):
            m = inf_t
            for t in range(_TPG):
                ti = g * _TPG + t
                tile = dist_ref[:, ti * 128:(ti + 1) * 128]
                valid = (tile > tlv) | ((tile == tlv) & (tl + ti * 128 > tli))
                m = jnp.minimum(m, jnp.where(valid, tile, jnp.inf))
            c = big_t
            for t in range(_TPG):
                ti = g * _TPG + t
                tile = dist_ref[:, ti * 128:(ti + 1) * 128]
                colt = tl + ti * 128
                valid = (tile > tlv) | ((tile == tlv) & (colt > tli))
                c = jnp.minimum(c, jnp.where(valid & (tile == m), colt, _BIGI))
            ms[rnd][:, sl] = m
            cs[rnd][:, sl] = c
            tlv, tli = m, c

    clsio = lax.broadcasted_iota(_I32, (_BR, _NCLS), 1)
    overflow = jnp.zeros((_BR, _NCLS), jnp.bool_)
    for k in range(_K):
        a = [ms[r][...] for r in range(_NSLOT)]
        b = [cs[r][...] for r in range(_NSLOT)]
        mrow = jnp.min(a[0], axis=1, keepdims=True)
        j = jnp.min(jnp.where(a[0] == mrow, b[0], _BIGI), axis=1,
                    keepdims=True)
        idx_ref[:, pl.ds(k, 1)] = j
        g = jnp.zeros_like(j)
        for t in range(1, _NG):
            g = g + (j >= t * (_TPG * 128)).astype(_I32)
        cls = g * 128 + (j & 127)
        iscls = clsio == cls
        overflow = overflow | (iscls & (a[1] == jnp.inf))
        for r in range(_NSLOT - 1):
            ms[r][...] = jnp.where(iscls, a[r + 1], a[r])
            cs[r][...] = jnp.where(iscls, b[r + 1], b[r])
        ms[_NSLOT - 1][...] = jnp.where(iscls, jnp.inf, a[_NSLOT - 1])
        cs[_NSLOT - 1][...] = jnp.where(iscls, _BIGI, b[_NSLOT - 1])

    bad = jnp.max(overflow.astype(_I32)) > 0

    @pl.when(bad)
    def _():
        lvf = jnp.full((_BR, 1), -jnp.inf, _F32)
        lif = jnp.full((_BR, 1), -1, _I32)
        for k in range(_K):
            d = dist_ref[...]
            validf = (d > lvf) | ((d == lvf) & (col > lif))
            candf = jnp.where(validf, d, jnp.inf)
            mf = jnp.min(candf, axis=1, keepdims=True)
            jf = jnp.min(jnp.where(candf == mf, col, _BIGI), axis=1,
                         keepdims=True)
            idx_ref[:, pl.ds(k, 1)] = jf
            lvf, lif = mf, jf


def _stageB(xpad, xt):
    def body(xr_ref, xt_ref, idx_ref, *scratch):
        ms = list(scratch[1:1 + _NSLOT])
        cs = list(scratch[1 + _NSLOT:])
        _stageB_body(xr_ref, xt_ref, idx_ref, scratch[0], ms, cs)

    return pl.pallas_call(
        body,
        grid=(_NBLK,),
        in_specs=[
            pl.BlockSpec((_BR, _D), lambda i: (i, 0)),
            pl.BlockSpec((_D, _NPC), lambda i: (0, 0)),
        ],
        out_specs=pl.BlockSpec((_BR, _K), lambda i: (i, 0)),
        out_shape=jax.ShapeDtypeStruct((_NPAD, _K), _I32),
        scratch_shapes=(
            [pltpu.VMEM((_BR, _NPC), _F32)]
            + [pltpu.VMEM((_BR, _NCLS), _F32) for _ in range(_NSLOT)]
            + [pltpu.VMEM((_BR, _NCLS), _I32) for _ in range(_NSLOT)]
        ),
    )(xpad, xt)


def _stageC(table, idx):
    mesh = plsc.VectorSubcoreMesh(core_axis_name="c", subcore_axis_name="s")

    @functools.partial(
        pl.kernel,
        mesh=mesh,
        out_type=jax.ShapeDtypeStruct((_EDGES, 256), _F32),
        scratch_types=[
            pltpu.VMEM((_E_PER_W,), _I32),
            pltpu.VMEM((_GCHUNK, 256), _F32),
            pltpu.SemaphoreType.DMA,
        ],
    )
    def k(table_hbm, idx_hbm, out_hbm, idx_v, rows_v, sem):
        wid = lax.axis_index("s") * _SC_NC + lax.axis_index("c")
        base = wid * _E_PER_W
        pltpu.sync_copy(idx_hbm.at[pl.ds(base, _E_PER_W)], idx_v)

        def body(t, _):
            off = pl.multiple_of(t * _GCHUNK, 8)
            pltpu.async_copy(
                table_hbm.at[idx_v.at[pl.ds(off, _GCHUNK)]], rows_v, sem
            ).wait()
            pltpu.sync_copy(rows_v, out_hbm.at[pl.ds(base + off, _GCHUNK)])
            return ()

        lax.fori_loop(0, _GITERS, body, (), unroll=False)

    return k(table, idx)


def _stageD_body(gath_ref, cq_ref, g_ref, xmax_ref, w1_ref, w2_ref, bm_ref,
                 out_ref):
    cq = cq_ref[...]
    g = g_ref[...]
    c = cq[:, :_D]
    q = cq[:, _D:]
    s_own = g[:, :_D]
    u_own = g[:, _D:]

    a_self = c - s_own
    m = a_self
    for k in range(_K):
        m = jnp.maximum(m, c - gath_ref[:, k * 256:k * 256 + _D])
    es = jnp.exp(a_self - m)
    den = es
    num = es * u_own
    for k in range(_K):
        blk = gath_ref[:, pl.ds(k * 256, 256)]
        e = jnp.exp((c - blk[:, :_D]) - m)
        den = den + e
        num = num + e * blk[:, _D:]
    h1 = (num + den * q) / (den + _F32(1e-16))

    r = lax.dot_general(xmax_ref[...], w2_ref[...], (((1,), (0,)), ((), ())),
                        precision=lax.Precision.HIGHEST) + bm_ref[...]
    o = lax.dot_general(h1, w1_ref[...], (((1,), (0,)), ((), ())),
                        precision=lax.Precision.HIGHEST) + r
    out_ref[...] = jnp.maximum(o, 0.0)


def _stageD(gath2, cq, g, xmax, w1, w2, bm):
    return pl.pallas_call(
        _stageD_body,
        grid=(_NBLK,),
        in_specs=[
            pl.BlockSpec((_BR, _K * 256), lambda i: (i, 0)),
            pl.BlockSpec((_BR, 256), lambda i: (i, 0)),
            pl.BlockSpec((_BR, 256), lambda i: (i, 0)),
            pl.BlockSpec((1, _D), lambda i: (0, 0)),
            pl.BlockSpec((_D, _D), lambda i: (0, 0)),
            pl.BlockSpec((_D, _D), lambda i: (0, 0)),
            pl.BlockSpec((1, _D), lambda i: (0, 0)),
        ],
        out_specs=pl.BlockSpec((_BR, _D), lambda i: (i, 0)),
        out_shape=jax.ShapeDtypeStruct((_N, _D), _F32),
    )(gath2, cq, g, xmax, w1, w2, bm)


def kernel(x, pos, W_lin, W_src, W_dst, W_pos, b_pos, W_mlp, b_mlp):
    xpad = jnp.zeros((_NPAD, _D), _F32).at[:_N].set(x)
    xt = jnp.zeros((_D, _NPC), _F32).at[:, :_N].set(x.T)

    pz = jnp.zeros((_DIN - _D - 3, 512), _F32)
    wx = jnp.concatenate([W_src, W_lin, W_dst, jnp.zeros((_D, _D), _F32)],
                         axis=1)
    wp = jnp.concatenate([W_pos, -W_pos, W_pos, W_pos], axis=1)
    wc = jnp.concatenate([wx, wp, pz], axis=0)
    zb = jnp.zeros((256,), _F32)
    bvec = jnp.concatenate([zb, b_pos, b_pos]).reshape(1, 512)

    xp_pad = jnp.zeros((_NPAD, _DIN), _F32)
    xp_pad = xp_pad.at[:_N, :_D].set(x).at[:_N, _D:_D + 3].set(pos)

    g, cq, xmax = _stageA(xp_pad, wc, bvec)
    idx = _stageB(xpad, xt)[:_N, :_K].reshape(-1)
    gath = _stageC(g, idx)
    gath2 = gath.reshape(_N, _K * 256)

    w1 = W_mlp[:_D, :]
    w2 = W_mlp[_D:, :]
    bm = b_mlp.reshape(1, _D)
    return _stageD(gath2, cq, g, xmax, w1, w2, bm)


# speedup vs baseline: 5.7598x; 1.0003x over previous
import functools

import jax
import jax.numpy as jnp
from jax import lax
from jax.experimental import pallas as pl
from jax.experimental.pallas import tpu as pltpu
from jax.experimental.pallas import tpu_sc as plsc

_N = 10000
_D = 128
_K = 32
_BR = 128
_NBLK = 79
_NPAD = _NBLK * _BR
_DIN = 136

_F32 = jnp.float32
_I32 = jnp.int32

_SC_NC = 2
_SC_NS = 16
_SC_NW = _SC_NC * _SC_NS
_EDGES = _N * _K
_E_PER_W = _EDGES // _SC_NW
_GCHUNK = 80
_GITERS = _E_PER_W // _GCHUNK


def _stageA_body(xp_ref, wc_ref, bvec_ref, g_ref, cq_ref, xmax_ref):
    i = pl.program_id(0)
    xb = xp_ref[...]
    y = lax.dot_general(xb, wc_ref[...], (((1,), (0,)), ((), ())),
                        precision=lax.Precision.HIGHEST)
    y = y + bvec_ref[...]
    g_ref[...] = y[:, :256]
    cq_ref[...] = y[:, 256:512]
    row = i * _BR + lax.broadcasted_iota(_I32, (_BR, 1), 0)
    xm = jnp.where(row < _N, xb[:, :_D], -jnp.inf)
    bmax = jnp.max(xm, axis=0, keepdims=True)

    @pl.when(i == 0)
    def _():
        xmax_ref[...] = jnp.full((1, _D), -jnp.inf, _F32)

    xmax_ref[...] = jnp.maximum(xmax_ref[...], bmax)


def _stageA(xp_pad, wc, bvec):
    return pl.pallas_call(
        _stageA_body,
        grid=(_NBLK,),
        in_specs=[
            pl.BlockSpec((_BR, _DIN), lambda i: (i, 0)),
            pl.BlockSpec((_DIN, 512), lambda i: (0, 0)),
            pl.BlockSpec((1, 512), lambda i: (0, 0)),
        ],
        out_specs=[
            pl.BlockSpec((_BR, 256), lambda i: (i, 0)),
            pl.BlockSpec((_BR, 256), lambda i: (i, 0)),
            pl.BlockSpec((1, _D), lambda i: (0, 0)),
        ],
        out_shape=[
            jax.ShapeDtypeStruct((_N, 256), _F32),
            jax.ShapeDtypeStruct((_N, 256), _F32),
            jax.ShapeDtypeStruct((1, _D), _F32),
        ],
    )(xp_pad, wc, bvec)


_TPG = 40
_NG = 2
_NPC = _NG * _TPG * 128
_NCLS = _NG * 128
_BIGI = 2**30


_NSLOT = 5


def _stageB_body(xr_ref, xt_ref, idx_ref, dist_ref, ms, cs):
    i = pl.program_id(0)
    xr = xr_ref[...]
    xt = xt_ref[...]
    sqi = jnp.sum(xr * xr, axis=1, keepdims=True)
    sqj = jnp.sum(xt * xt, axis=0, keepdims=True)
    mm = lax.dot_general(xr, xt, (((1,), (0,)), ((), ())))
    col = lax.broadcasted_iota(_I32, (_BR, _NPC), 1)
    row = i * _BR + lax.broadcasted_iota(_I32, (_BR, _NPC), 0)
    dist = (sqi + sqj) - 2.0 * mm
    dist = dist + jnp.where(col == row, _F32(1e10), _F32(0.0))
    dist = jnp.where(col >= _N, jnp.inf, dist)
    dist_ref[...] = dist

    tl = lax.broadcasted_iota(_I32, (_BR, 128), 1)
    inf_t = jnp.full((_BR, 128), jnp.inf, _F32)
    big_t = jnp.full((_BR, 128), _BIGI, _I32)

    for g in range(_NG):
        sl = pl.ds(g * 128, 128)
        tlv = jnp.full((_BR, 128), -jnp.inf, _F32)
        tli = jnp.full((_BR, 128), -1, _I32)
        for rnd in range(_NSLOT):
            m = inf_t
            for t in range(_TPG):
                ti = g * _TPG + t
                tile = dist_ref[:, ti * 128:(ti + 1) * 128]
                valid = (tile > tlv) | ((tile == tlv) & (tl + ti * 128 > tli))
                m = jnp.minimum(m, jnp.where(valid, tile, jnp.inf))
            c = big_t
            for t in range(_TPG):
                ti = g * _TPG + t
                tile = dist_ref[:, ti * 128:(ti + 1) * 128]
                colt = tl + ti * 128
                valid = (tile > tlv) | ((tile == tlv) & (colt > tli))
                c = jnp.minimum(c, jnp.where(valid & (tile == m), colt, _BIGI))
            ms[rnd][:, sl] = m
            cs[rnd][:, sl] = c
            tlv, tli = m, c

    clsio = lax.broadcasted_iota(_I32, (_BR, _NCLS), 1)
    overflow = jnp.zeros((_BR, _NCLS), jnp.bool_)
    a = [ms[r][...] for r in range(_NSLOT)]
    b = [cs[r][...] for r in range(_NSLOT)]
    for k in range(_K):
        mrow = jnp.min(a[0], axis=1, keepdims=True)
        j = jnp.min(jnp.where(a[0] == mrow, b[0], _BIGI), axis=1,
                    keepdims=True)
        idx_ref[:, pl.ds(k, 1)] = j
        g = jnp.zeros_like(j)
        for t in range(1, _NG):
            g = g + (j >= t * (_TPG * 128)).astype(_I32)
        cls = g * 128 + (j & 127)
        iscls = clsio == cls
        overflow = overflow | (iscls & (a[1] == jnp.inf))
        a = ([jnp.where(iscls, a[r + 1], a[r]) for r in range(_NSLOT - 1)]
             + [jnp.where(iscls, jnp.inf, a[_NSLOT - 1])])
        b = ([jnp.where(iscls, b[r + 1], b[r]) for r in range(_NSLOT - 1)]
             + [jnp.where(iscls, _BIGI, b[_NSLOT - 1])])

    bad = jnp.max(overflow.astype(_I32)) > 0

    @pl.when(bad)
    def _():
        lvf = jnp.full((_BR, 1), -jnp.inf, _F32)
        lif = jnp.full((_BR, 1), -1, _I32)
        for k in range(_K):
            d = dist_ref[...]
            validf = (d > lvf) | ((d == lvf) & (col > lif))
            candf = jnp.where(validf, d, jnp.inf)
            mf = jnp.min(candf, axis=1, keepdims=True)
            jf = jnp.min(jnp.where(candf == mf, col, _BIGI), axis=1,
                         keepdims=True)
            idx_ref[:, pl.ds(k, 1)] = jf
            lvf, lif = mf, jf


def _stageB(xpad, xt):
    def body(xr_ref, xt_ref, idx_ref, *scratch):
        ms = list(scratch[1:1 + _NSLOT])
        cs = list(scratch[1 + _NSLOT:])
        _stageB_body(xr_ref, xt_ref, idx_ref, scratch[0], ms, cs)

    return pl.pallas_call(
        body,
        grid=(_NBLK,),
        in_specs=[
            pl.BlockSpec((_BR, _D), lambda i: (i, 0)),
            pl.BlockSpec((_D, _NPC), lambda i: (0, 0)),
        ],
        out_specs=pl.BlockSpec((_BR, _K), lambda i: (i, 0)),
        out_shape=jax.ShapeDtypeStruct((_NPAD, _K), _I32),
        scratch_shapes=(
            [pltpu.VMEM((_BR, _NPC), _F32)]
            + [pltpu.VMEM((_BR, _NCLS), _F32) for _ in range(_NSLOT)]
            + [pltpu.VMEM((_BR, _NCLS), _I32) for _ in range(_NSLOT)]
        ),
    )(xpad, xt)


def _stageC(table, idx):
    mesh = plsc.VectorSubcoreMesh(core_axis_name="c", subcore_axis_name="s")

    @functools.partial(
        pl.kernel,
        mesh=mesh,
        out_type=jax.ShapeDtypeStruct((_EDGES, 256), _F32),
        scratch_types=[
            pltpu.VMEM((_E_PER_W,), _I32),
            pltpu.VMEM((_GCHUNK, 256), _F32),
            pltpu.SemaphoreType.DMA,
        ],
    )
    def k(table_hbm, idx_hbm, out_hbm, idx_v, rows_v, sem):
        wid = lax.axis_index("s") * _SC_NC + lax.axis_index("c")
        base = wid * _E_PER_W
        pltpu.sync_copy(idx_hbm.at[pl.ds(base, _E_PER_W)], idx_v)

        def body(t, _):
            off = pl.multiple_of(t * _GCHUNK, 8)
            pltpu.async_copy(
                table_hbm.at[idx_v.at[pl.ds(off, _GCHUNK)]], rows_v, sem
            ).wait()
            pltpu.sync_copy(rows_v, out_hbm.at[pl.ds(base + off, _GCHUNK)])
            return ()

        lax.fori_loop(0, _GITERS, body, (), unroll=False)

    return k(table, idx)


def _stageD_body(gath_ref, cq_ref, g_ref, xmax_ref, w1_ref, w2_ref, bm_ref,
                 out_ref):
    cq = cq_ref[...]
    g = g_ref[...]
    c = cq[:, :_D]
    q = cq[:, _D:]
    s_own = g[:, :_D]
    u_own = g[:, _D:]

    a_self = c - s_own
    m = a_self
    for k in range(_K):
        m = jnp.maximum(m, c - gath_ref[:, k * 256:k * 256 + _D])
    es = jnp.exp(a_self - m)
    den = es
    num = es * u_own
    for k in range(_K):
        blk = gath_ref[:, pl.ds(k * 256, 256)]
        e = jnp.exp((c - blk[:, :_D]) - m)
        den = den + e
        num = num + e * blk[:, _D:]
    h1 = (num + den * q) / (den + _F32(1e-16))

    r = lax.dot_general(xmax_ref[...], w2_ref[...], (((1,), (0,)), ((), ())),
                        precision=lax.Precision.HIGHEST) + bm_ref[...]
    o = lax.dot_general(h1, w1_ref[...], (((1,), (0,)), ((), ())),
                        precision=lax.Precision.HIGHEST) + r
    out_ref[...] = jnp.maximum(o, 0.0)


def _stageD(gath2, cq, g, xmax, w1, w2, bm):
    return pl.pallas_call(
        _stageD_body,
        grid=(_NBLK,),
        in_specs=[
            pl.BlockSpec((_BR, _K * 256), lambda i: (i, 0)),
            pl.BlockSpec((_BR, 256), lambda i: (i, 0)),
            pl.BlockSpec((_BR, 256), lambda i: (i, 0)),
            pl.BlockSpec((1, _D), lambda i: (0, 0)),
            pl.BlockSpec((_D, _D), lambda i: (0, 0)),
            pl.BlockSpec((_D, _D), lambda i: (0, 0)),
            pl.BlockSpec((1, _D), lambda i: (0, 0)),
        ],
        out_specs=pl.BlockSpec((_BR, _D), lambda i: (i, 0)),
        out_shape=jax.ShapeDtypeStruct((_N, _D), _F32),
    )(gath2, cq, g, xmax, w1, w2, bm)


def kernel(x, pos, W_lin, W_src, W_dst, W_pos, b_pos, W_mlp, b_mlp):
    xpad = jnp.zeros((_NPAD, _D), _F32).at[:_N].set(x)
    xt = jnp.zeros((_D, _NPC), _F32).at[:, :_N].set(x.T)

    pz = jnp.zeros((_DIN - _D - 3, 512), _F32)
    wx = jnp.concatenate([W_src, W_lin, W_dst, jnp.zeros((_D, _D), _F32)],
                         axis=1)
    wp = jnp.concatenate([W_pos, -W_pos, W_pos, W_pos], axis=1)
    wc = jnp.concatenate([wx, wp, pz], axis=0)
    zb = jnp.zeros((256,), _F32)
    bvec = jnp.concatenate([zb, b_pos, b_pos]).reshape(1, 512)

    xp_pad = jnp.zeros((_NPAD, _DIN), _F32)
    xp_pad = xp_pad.at[:_N, :_D].set(x).at[:_N, _D:_D + 3].set(pos)

    g, cq, xmax = _stageA(xp_pad, wc, bvec)
    idx = _stageB(xpad, xt)[:_N, :_K].reshape(-1)
    gath = _stageC(g, idx)
    gath2 = gath.reshape(_N, _K * 256)

    w1 = W_mlp[:_D, :]
    w2 = W_mlp[_D:, :]
    bm = b_mlp.reshape(1, _D)
    return _stageD(gath2, cq, g, xmax, w1, w2, bm)


# speedup vs baseline: 5.8301x; 1.0122x over previous
import functools

import jax
import jax.numpy as jnp
from jax import lax
from jax.experimental import pallas as pl
from jax.experimental.pallas import tpu as pltpu
from jax.experimental.pallas import tpu_sc as plsc

_N = 10000
_D = 128
_K = 32
_BR = 128
_NBLK = 79
_NPAD = _NBLK * _BR
_DIN = 136

_F32 = jnp.float32
_I32 = jnp.int32

_SC_NC = 2
_SC_NS = 16
_SC_NW = _SC_NC * _SC_NS
_EDGES = _N * _K
_E_PER_W = _EDGES // _SC_NW
_GCHUNK = 80
_GITERS = _E_PER_W // _GCHUNK


def _stageA_body(xp_ref, wc_ref, bvec_ref, g_ref, cq_ref, xmax_ref):
    i = pl.program_id(0)
    xb = xp_ref[...]
    y = lax.dot_general(xb, wc_ref[...], (((1,), (0,)), ((), ())),
                        precision=lax.Precision.HIGHEST)
    y = y + bvec_ref[...]
    g_ref[...] = y[:, :256]
    cq_ref[...] = y[:, 256:512]
    row = i * _BR + lax.broadcasted_iota(_I32, (_BR, 1), 0)
    xm = jnp.where(row < _N, xb[:, :_D], -jnp.inf)
    bmax = jnp.max(xm, axis=0, keepdims=True)

    @pl.when(i == 0)
    def _():
        xmax_ref[...] = jnp.full((1, _D), -jnp.inf, _F32)

    xmax_ref[...] = jnp.maximum(xmax_ref[...], bmax)


def _stageA(xp_pad, wc, bvec):
    return pl.pallas_call(
        _stageA_body,
        grid=(_NBLK,),
        in_specs=[
            pl.BlockSpec((_BR, _DIN), lambda i: (i, 0)),
            pl.BlockSpec((_DIN, 512), lambda i: (0, 0)),
            pl.BlockSpec((1, 512), lambda i: (0, 0)),
        ],
        out_specs=[
            pl.BlockSpec((_BR, 256), lambda i: (i, 0)),
            pl.BlockSpec((_BR, 256), lambda i: (i, 0)),
            pl.BlockSpec((1, _D), lambda i: (0, 0)),
        ],
        out_shape=[
            jax.ShapeDtypeStruct((_N, 256), _F32),
            jax.ShapeDtypeStruct((_N, 256), _F32),
            jax.ShapeDtypeStruct((1, _D), _F32),
        ],
    )(xp_pad, wc, bvec)


_TPG = 40
_NG = 2
_NPC = _NG * _TPG * 128
_NCLS = _NG * 128
_BIGI = 2**30


_NSLOT = 5


def _stageB_body(xr_ref, xt_ref, idx_ref, dist_ref, ms, cs):
    i = pl.program_id(0)
    xr = xr_ref[...]
    xt = xt_ref[...]
    sqi = jnp.sum(xr * xr, axis=1, keepdims=True)
    sqj = jnp.sum(xt * xt, axis=0, keepdims=True)
    mm = lax.dot_general(xr, xt, (((1,), (0,)), ((), ())))
    col = lax.broadcasted_iota(_I32, (_BR, _NPC), 1)
    row = i * _BR + lax.broadcasted_iota(_I32, (_BR, _NPC), 0)
    dist = (sqi + sqj) - 2.0 * mm
    dist = dist + jnp.where(col == row, _F32(1e10), _F32(0.0))
    dist = jnp.where(col >= _N, jnp.inf, dist)
    dist_ref[...] = dist

    tl = lax.broadcasted_iota(_I32, (_BR, 128), 1)
    inf_t = jnp.full((_BR, 128), jnp.inf, _F32)
    big_t = jnp.full((_BR, 128), _BIGI, _I32)

    for g in range(_NG):
        sl = pl.ds(g * 128, 128)
        tlv = jnp.full((_BR, 128), -jnp.inf, _F32)
        tli = jnp.full((_BR, 128), -1, _I32)
        for rnd in range(_NSLOT):
            m = inf_t
            for t in range(_TPG):
                ti = g * _TPG + t
                tile = dist_ref[:, ti * 128:(ti + 1) * 128]
                valid = (tile > tlv) | ((tile == tlv) & (tl + ti * 128 > tli))
                m = jnp.minimum(m, jnp.where(valid, tile, jnp.inf))
            c = big_t
            for t in range(_TPG):
                ti = g * _TPG + t
                tile = dist_ref[:, ti * 128:(ti + 1) * 128]
                colt = tl + ti * 128
                valid = (tile > tlv) | ((tile == tlv) & (colt > tli))
                c = jnp.minimum(c, jnp.where(valid & (tile == m), colt, _BIGI))
            ms[rnd][:, sl] = m
            cs[rnd][:, sl] = c
            tlv, tli = m, c

    clsio = lax.broadcasted_iota(_I32, (_BR, _NCLS), 1)
    overflow = jnp.zeros((_BR, _NCLS), jnp.bool_)
    a = [ms[r][...] for r in range(_NSLOT)]
    b = [cs[r][...] for r in range(_NSLOT)]
    for k in range(_K):
        mrow = jnp.min(a[0], axis=1, keepdims=True)
        j = jnp.min(jnp.where(a[0] == mrow, b[0], _BIGI), axis=1,
                    keepdims=True)
        idx_ref[:, pl.ds(k, 1)] = j
        g = jnp.zeros_like(j)
        for t in range(1, _NG):
            g = g + (j >= t * (_TPG * 128)).astype(_I32)
        cls = g * 128 + (j & 127)
        iscls = clsio == cls
        overflow = overflow | (iscls & (a[1] == jnp.inf))
        a = ([jnp.where(iscls, a[r + 1], a[r]) for r in range(_NSLOT - 1)]
             + [jnp.where(iscls, jnp.inf, a[_NSLOT - 1])])
        b = ([jnp.where(iscls, b[r + 1], b[r]) for r in range(_NSLOT - 1)]
             + [jnp.where(iscls, _BIGI, b[_NSLOT - 1])])

    bad = jnp.max(overflow.astype(_I32)) > 0

    @pl.when(bad)
    def _():
        lvf = jnp.full((_BR, 1), -jnp.inf, _F32)
        lif = jnp.full((_BR, 1), -1, _I32)
        for k in range(_K):
            d = dist_ref[...]
            validf = (d > lvf) | ((d == lvf) & (col > lif))
            candf = jnp.where(validf, d, jnp.inf)
            mf = jnp.min(candf, axis=1, keepdims=True)
            jf = jnp.min(jnp.where(candf == mf, col, _BIGI), axis=1,
                         keepdims=True)
            idx_ref[:, pl.ds(k, 1)] = jf
            lvf, lif = mf, jf


def _stageB(xpad, xt):
    def body(xr_ref, xt_ref, idx_ref, *scratch):
        ms = list(scratch[1:1 + _NSLOT])
        cs = list(scratch[1 + _NSLOT:])
        _stageB_body(xr_ref, xt_ref, idx_ref, scratch[0], ms, cs)

    return pl.pallas_call(
        body,
        grid=(_NBLK,),
        in_specs=[
            pl.BlockSpec((_BR, _D), lambda i: (i, 0)),
            pl.BlockSpec((_D, _NPC), lambda i: (0, 0)),
        ],
        out_specs=pl.BlockSpec((_BR, _K), lambda i: (i, 0)),
        out_shape=jax.ShapeDtypeStruct((_NPAD, _K), _I32),
        scratch_shapes=(
            [pltpu.VMEM((_BR, _NPC), _F32)]
            + [pltpu.VMEM((_BR, _NCLS), _F32) for _ in range(_NSLOT)]
            + [pltpu.VMEM((_BR, _NCLS), _I32) for _ in range(_NSLOT)]
        ),
    )(xpad, xt)


def _stageC(table, idx):
    mesh = plsc.VectorSubcoreMesh(core_axis_name="c", subcore_axis_name="s")

    @functools.partial(
        pl.kernel,
        mesh=mesh,
        out_type=jax.ShapeDtypeStruct((_EDGES, 256), _F32),
        scratch_types=[
            pltpu.VMEM((_E_PER_W,), _I32),
            pltpu.VMEM((_GCHUNK, 256), _F32),
            pltpu.VMEM((_GCHUNK, 256), _F32),
            pltpu.SemaphoreType.DMA,
            pltpu.SemaphoreType.DMA,
        ],
    )
    def k(table_hbm, idx_hbm, out_hbm, idx_v, buf_a, buf_b, sem_a, sem_b):
        wid = lax.axis_index("s") * _SC_NC + lax.axis_index("c")
        base = wid * _E_PER_W
        pltpu.sync_copy(idx_hbm.at[pl.ds(base, _E_PER_W)], idx_v)

        def start(c, buf, sem):
            off = pl.multiple_of(c * _GCHUNK, 8)
            pltpu.make_async_copy(
                table_hbm.at[idx_v.at[pl.ds(off, _GCHUNK)]], buf, sem
            ).start()

        def wait(buf, sem):
            pltpu.make_async_copy(
                out_hbm.at[pl.ds(base, _GCHUNK)], buf, sem
            ).wait()

        def put(c, buf):
            off = pl.multiple_of(c * _GCHUNK, 8)
            pltpu.sync_copy(buf, out_hbm.at[pl.ds(base + off, _GCHUNK)])

        start(0, buf_a, sem_a)
        start(1, buf_b, sem_b)

        def body(i, _):
            c = i * 2
            wait(buf_a, sem_a)
            put(c, buf_a)
            start(c + 2, buf_a, sem_a)
            wait(buf_b, sem_b)
            put(c + 1, buf_b)
            start(c + 3, buf_b, sem_b)
            return ()

        lax.fori_loop(0, (_GITERS - 3) // 2, body, (), unroll=False)

        wait(buf_a, sem_a)
        put(_GITERS - 3, buf_a)
        start(_GITERS - 1, buf_a, sem_a)
        wait(buf_b, sem_b)
        put(_GITERS - 2, buf_b)
        wait(buf_a, sem_a)
        put(_GITERS - 1, buf_a)

    return k(table, idx)


def _stageD_body(gath_ref, cq_ref, g_ref, xmax_ref, w1_ref, w2_ref, bm_ref,
                 out_ref):
    cq = cq_ref[...]
    g = g_ref[...]
    c = cq[:, :_D]
    q = cq[:, _D:]
    s_own = g[:, :_D]
    u_own = g[:, _D:]

    a_self = c - s_own
    m = a_self
    for k in range(_K):
        m = jnp.maximum(m, c - gath_ref[:, k * 256:k * 256 + _D])
    es = jnp.exp(a_self - m)
    den = es
    num = es * u_own
    for k in range(_K):
        blk = gath_ref[:, pl.ds(k * 256, 256)]
        e = jnp.exp((c - blk[:, :_D]) - m)
        den = den + e
        num = num + e * blk[:, _D:]
    h1 = (num + den * q) / (den + _F32(1e-16))

    r = lax.dot_general(xmax_ref[...], w2_ref[...], (((1,), (0,)), ((), ())),
                        precision=lax.Precision.HIGHEST) + bm_ref[...]
    o = lax.dot_general(h1, w1_ref[...], (((1,), (0,)), ((), ())),
                        precision=lax.Precision.HIGHEST) + r
    out_ref[...] = jnp.maximum(o, 0.0)


def _stageD(gath2, cq, g, xmax, w1, w2, bm):
    return pl.pallas_call(
        _stageD_body,
        grid=(_NBLK,),
        in_specs=[
            pl.BlockSpec((_BR, _K * 256), lambda i: (i, 0)),
            pl.BlockSpec((_BR, 256), lambda i: (i, 0)),
            pl.BlockSpec((_BR, 256), lambda i: (i, 0)),
            pl.BlockSpec((1, _D), lambda i: (0, 0)),
            pl.BlockSpec((_D, _D), lambda i: (0, 0)),
            pl.BlockSpec((_D, _D), lambda i: (0, 0)),
            pl.BlockSpec((1, _D), lambda i: (0, 0)),
        ],
        out_specs=pl.BlockSpec((_BR, _D), lambda i: (i, 0)),
        out_shape=jax.ShapeDtypeStruct((_N, _D), _F32),
    )(gath2, cq, g, xmax, w1, w2, bm)


def kernel(x, pos, W_lin, W_src, W_dst, W_pos, b_pos, W_mlp, b_mlp):
    xpad = jnp.zeros((_NPAD, _D), _F32).at[:_N].set(x)
    xt = jnp.zeros((_D, _NPC), _F32).at[:, :_N].set(x.T)

    pz = jnp.zeros((_DIN - _D - 3, 512), _F32)
    wx = jnp.concatenate([W_src, W_lin, W_dst, jnp.zeros((_D, _D), _F32)],
                         axis=1)
    wp = jnp.concatenate([W_pos, -W_pos, W_pos, W_pos], axis=1)
    wc = jnp.concatenate([wx, wp, pz], axis=0)
    zb = jnp.zeros((256,), _F32)
    bvec = jnp.concatenate([zb, b_pos, b_pos]).reshape(1, 512)

    xp_pad = jnp.zeros((_NPAD, _DIN), _F32)
    xp_pad = xp_pad.at[:_N, :_D].set(x).at[:_N, _D:_D + 3].set(pos)

    g, cq, xmax = _stageA(xp_pad, wc, bvec)
    idx = _stageB(xpad, xt)[:_N, :_K].reshape(-1)
    gath = _stageC(g, idx)
    gath2 = gath.reshape(_N, _K * 256)

    w1 = W_mlp[:_D, :]
    w2 = W_mlp[_D:, :]
    bm = b_mlp.reshape(1, _D)
    return _stageD(gath2, cq, g, xmax, w1, w2, bm)
